# baseline (speedup 1.0000x reference)
import functools

import jax
import jax.numpy as jnp
from jax import lax
from jax.experimental import pallas as pl
from jax.experimental.pallas import tpu as pltpu
from jax.experimental.pallas import tpu_sc as plsc

_N_TRAIN = 100000
_DIM = 64
_BATCH = 1024
_K = 16
_N_CLASSES = 10

_RB = 512
_BT = 512
_NBT = (_N_TRAIN + _BT - 1) // _BT
_NTP = _NBT * _BT
_SB = 512

_INF = float("inf")
_BIGI = 0x7FFFFFFF
_BIGF = 3.0e8


def _topk_body(x_ref, t_ref, vals_ref, idx_ref, topv, topi, dscr, x2s):
    j = pl.program_id(1)

    @pl.when(j == 0)
    def _():
        topv[...] = jnp.full((_RB, _K), _INF, jnp.float32)
        topi[...] = jnp.zeros((_RB, _K), jnp.int32)
        xb0 = x_ref[...]
        x2s[...] = jnp.sum(xb0 * xb0, axis=1, keepdims=True)

    xb = x_ref[...]
    tb = t_ref[...]
    x2 = x2s[...]
    t2 = jnp.sum(tb * tb, axis=0, keepdims=True)
    xt = jnp.dot(xb, tb, preferred_element_type=jnp.float32,
                 precision=lax.Precision.DEFAULT)
    sq = x2 + t2 - 2.0 * xt

    gcol = (jnp.float32(j * _BT)
            + lax.broadcasted_iota(jnp.int32, (_RB, _BT), 1)
            .astype(jnp.float32))
    dscr[...] = jnp.sqrt(jnp.maximum(sq, 1e-12))
    d = dscr[...]

    thresh = topv[...][:, _K - 1:_K]
    cnt = jnp.sum(jnp.where(d < thresh, 1.0, 0.0), axis=1)
    npass = jnp.minimum(jnp.max(cnt), jnp.float32(_K))

    k_iota = lax.broadcasted_iota(jnp.int32, (_RB, _K), 1)
    for t in range(_K):
        @pl.when(t < npass)
        def _():
            dd = dscr[...]
            m = jnp.min(dd, axis=1, keepdims=True)
            a = jnp.min(jnp.where(dd == m, gcol, _BIGF), axis=1,
                        keepdims=True)
            dscr[...] = jnp.where(gcol == a, _INF, dd)
            a_i = a.astype(jnp.int32)
            v = topv[...]
            ii = topi[...]
            pos = jnp.sum((v <= m).astype(jnp.int32), axis=1, keepdims=True)
            vprev = jnp.concatenate(
                [jnp.full((_RB, 1), _INF, jnp.float32), v[:, :_K - 1]],
                axis=1)
            iprev = jnp.concatenate(
                [jnp.zeros((_RB, 1), jnp.int32), ii[:, :_K - 1]], axis=1)
            topv[...] = jnp.where(
                k_iota < pos, v, jnp.where(k_iota == pos, m, vprev))
            topi[...] = jnp.where(
                k_iota < pos, ii, jnp.where(k_iota == pos, a_i, iprev))

    @pl.when(j == _NBT - 1)
    def _():
        vals_ref[...] = topv[...]
        idx_ref[...] = topi[...]


def _run_topk(x, td_t):
    return pl.pallas_call(
        _topk_body,
        grid=(_BATCH // _RB, _NBT),
        in_specs=[
            pl.BlockSpec((_RB, _DIM), lambda i, j: (i, 0)),
            pl.BlockSpec((_DIM, _BT), lambda i, j: (0, j)),
        ],
        out_specs=[
            pl.BlockSpec((_RB, _K), lambda i, j: (i, 0)),
            pl.BlockSpec((_RB, _K), lambda i, j: (i, 0)),
        ],
        out_shape=[
            jax.ShapeDtypeStruct((_BATCH, _K), jnp.float32),
            jax.ShapeDtypeStruct((_BATCH, _K), jnp.int32),
        ],
        scratch_shapes=[
            pltpu.VMEM((_RB, _K), jnp.float32),
            pltpu.VMEM((_RB, _K), jnp.int32),
            pltpu.VMEM((_RB, _BT), jnp.float32),
            pltpu.VMEM((_RB, 1), jnp.float32),
        ],
        compiler_params=pltpu.CompilerParams(
            dimension_semantics=("parallel", "arbitrary")),
    )(x, td_t)


_NW = 32
_GPW = _BATCH * _K // _NW
_CH = 128
_NCH = _GPW // _CH


def _sc_gather_body(labels_hbm, idx_hbm, out_hbm, idx_v, lab_v, sem):
    c = lax.axis_index("c")
    s = lax.axis_index("s")
    wid = s * 2 + c
    pltpu.sync_copy(idx_hbm.at[wid], idx_v)
    for t in range(_NCH):
        pltpu.async_copy(labels_hbm.at[idx_v.at[t]], lab_v.at[t], sem).wait()
    pltpu.sync_copy(lab_v, out_hbm.at[wid])


def _run_sc_gather(train_labels, idx):
    idx3 = idx.reshape(_NW, _NCH, _CH)
    mesh = plsc.VectorSubcoreMesh(core_axis_name="c", subcore_axis_name="s")
    kern = pl.kernel(
        _sc_gather_body,
        out_type=jax.ShapeDtypeStruct((_NW, _NCH, _CH), jnp.int32),
        mesh=mesh,
        scratch_types=[
            pltpu.VMEM((_NCH, _CH), jnp.int32),
            pltpu.VMEM((_NCH, _CH), jnp.int32),
            pltpu.SemaphoreType.DMA,
        ],
    )
    return kern(train_labels, idx3).reshape(_BATCH, _K)


def _vote_body(vals_ref, lab_ref, pred_ref, proba_ref):
    d = vals_ref[...]
    lab = lab_ref[...]
    w = 1.0 / d
    cls = lax.broadcasted_iota(jnp.int32, (_BATCH, _N_CLASSES), 1)
    acc = jnp.zeros((_BATCH, _N_CLASSES), jnp.float32)
    for k in range(_K):
        acc = acc + jnp.where(cls == lab[:, k:k + 1], w[:, k:k + 1], 0.0)
    s = jnp.sum(acc, axis=1, keepdims=True)
    s = jnp.where(s == 0.0, 1.0, s)
    proba = acc / s
    proba_ref[...] = proba
    m = jnp.max(proba, axis=1, keepdims=True)
    pred_ref[...] = jnp.min(
        jnp.where(proba == m, cls, _N_CLASSES), axis=1,
        keepdims=True)


def _run_vote(vals, labels):
    return pl.pallas_call(
        _vote_body,
        out_shape=[
            jax.ShapeDtypeStruct((_BATCH, 1), jnp.int32),
            jax.ShapeDtypeStruct((_BATCH, _N_CLASSES), jnp.float32),
        ],
    )(vals, labels)


def kernel(x, train_data, train_labels):
    td_t = jnp.pad(train_data, ((0, _NTP - _N_TRAIN), (0, 0)),
                   constant_values=1e18).T
    vals, idx = _run_topk(x, td_t)
    labels = _run_sc_gather(train_labels, idx)
    pred, proba = _run_vote(vals, labels)
    return (pred.reshape(_BATCH), proba)


# speedup vs baseline: 1.1682x; 1.1682x over previous
import jax
import jax.numpy as jnp
from jax import lax
from jax.experimental import pallas as pl
from jax.experimental.pallas import tpu as pltpu
from jax.experimental.pallas import tpu_sc as plsc

_N_TRAIN = 100000
_DIM = 64
_BATCH = 1024
_K = 16
_N_CLASSES = 10

_RB = 512
_BT = 1024
_NBT = (_N_TRAIN + _BT - 1) // _BT
_NTP = _NBT * _BT

_INF = float("inf")
_BIGF = 3.0e8


def _topk_body(x_ref, t_ref, vals_ref, idx_ref, topv, topi, dscr, x2s):
    j = pl.program_id(1)

    @pl.when(j == 0)
    def _():
        topv[...] = jnp.full((_RB, _K), _INF, jnp.float32)
        topi[...] = jnp.zeros((_RB, _K), jnp.int32)
        xb0 = x_ref[...]
        x2s[...] = jnp.sum(xb0 * xb0, axis=1, keepdims=True)

    xb = x_ref[...]
    tb = t_ref[...]
    x2 = x2s[...]
    t2 = jnp.sum(tb * tb, axis=0, keepdims=True)
    xt = jnp.dot(xb, tb, preferred_element_type=jnp.float32,
                 precision=lax.Precision.DEFAULT)
    sq = x2 + t2 - 2.0 * xt

    gcol = (jnp.float32(j * _BT)
            + lax.broadcasted_iota(jnp.int32, (_RB, _BT), 1)
            .astype(jnp.float32))
    dscr[...] = jnp.sqrt(jnp.maximum(sq, 1e-12))
    d = dscr[...]

    thresh = topv[...][:, _K - 1:_K]
    cnt = jnp.sum(jnp.where(d < thresh, 1.0, 0.0), axis=1)
    npass = jnp.minimum(jnp.max(cnt), jnp.float32(_K))

    k_iota = lax.broadcasted_iota(jnp.int32, (_RB, _K), 1)
    for t in range(_K):
        @pl.when(t < npass)
        def _():
            dd = dscr[...]
            m = jnp.min(dd, axis=1, keepdims=True)
            a = jnp.min(jnp.where(dd == m, gcol, _BIGF), axis=1,
                        keepdims=True)
            dscr[...] = jnp.where(gcol == a, _INF, dd)
            a_i = a.astype(jnp.int32)
            v = topv[...]
            ii = topi[...]
            pos = jnp.sum((v <= m).astype(jnp.int32), axis=1, keepdims=True)
            vprev = jnp.concatenate(
                [jnp.full((_RB, 1), _INF, jnp.float32), v[:, :_K - 1]],
                axis=1)
            iprev = jnp.concatenate(
                [jnp.zeros((_RB, 1), jnp.int32), ii[:, :_K - 1]], axis=1)
            topv[...] = jnp.where(
                k_iota < pos, v, jnp.where(k_iota == pos, m, vprev))
            topi[...] = jnp.where(
                k_iota < pos, ii, jnp.where(k_iota == pos, a_i, iprev))

    @pl.when(j == _NBT - 1)
    def _():
        vals_ref[...] = topv[...]
        idx_ref[...] = topi[...]


def _run_topk(x, td_t):
    return pl.pallas_call(
        _topk_body,
        grid=(_BATCH // _RB, _NBT),
        in_specs=[
            pl.BlockSpec((_RB, _DIM), lambda i, j: (i, 0)),
            pl.BlockSpec((_DIM, _BT), lambda i, j: (0, j)),
        ],
        out_specs=[
            pl.BlockSpec((_RB, _K), lambda i, j: (i, 0)),
            pl.BlockSpec((_RB, _K), lambda i, j: (i, 0)),
        ],
        out_shape=[
            jax.ShapeDtypeStruct((_BATCH, _K), jnp.float32),
            jax.ShapeDtypeStruct((_BATCH, _K), jnp.int32),
        ],
        scratch_shapes=[
            pltpu.VMEM((_RB, _K), jnp.float32),
            pltpu.VMEM((_RB, _K), jnp.int32),
            pltpu.VMEM((_RB, _BT), jnp.float32),
            pltpu.VMEM((_RB, 1), jnp.float32),
        ],
        compiler_params=pltpu.CompilerParams(
            dimension_semantics=("parallel", "arbitrary")),
    )(x, td_t)


_NW = 32
_GPW = _BATCH * _K // _NW
_CH = 128
_NCH = _GPW // _CH


def _sc_gather_body(labels_hbm, idx_hbm, out_hbm, idx_v, lab_v, sem):
    c = lax.axis_index("c")
    s = lax.axis_index("s")
    wid = s * 2 + c
    pltpu.sync_copy(idx_hbm.at[wid], idx_v)
    for t in range(_NCH):
        pltpu.async_copy(labels_hbm.at[idx_v.at[t]], lab_v.at[t], sem).wait()
    pltpu.sync_copy(lab_v, out_hbm.at[wid])


def _run_sc_gather(train_labels, idx):
    idx3 = idx.reshape(_NW, _NCH, _CH)
    mesh = plsc.VectorSubcoreMesh(core_axis_name="c", subcore_axis_name="s")
    kern = pl.kernel(
        _sc_gather_body,
        out_type=jax.ShapeDtypeStruct((_NW, _NCH, _CH), jnp.int32),
        mesh=mesh,
        scratch_types=[
            pltpu.VMEM((_NCH, _CH), jnp.int32),
            pltpu.VMEM((_NCH, _CH), jnp.int32),
            pltpu.SemaphoreType.DMA,
        ],
    )
    return kern(train_labels, idx3).reshape(_BATCH, _K)


def _vote_body(vals_ref, lab_ref, pred_ref, proba_ref):
    d = vals_ref[...]
    lab = lab_ref[...]
    w = 1.0 / d
    cls = lax.broadcasted_iota(jnp.int32, (_BATCH, _N_CLASSES), 1)
    acc = jnp.zeros((_BATCH, _N_CLASSES), jnp.float32)
    for k in range(_K):
        acc = acc + jnp.where(cls == lab[:, k:k + 1], w[:, k:k + 1], 0.0)
    s = jnp.sum(acc, axis=1, keepdims=True)
    s = jnp.where(s == 0.0, 1.0, s)
    proba = acc / s
    proba_ref[...] = proba
    m = jnp.max(proba, axis=1, keepdims=True)
    pred_ref[...] = jnp.min(
        jnp.where(proba == m, cls, _N_CLASSES), axis=1,
        keepdims=True)


def _run_vote(vals, labels):
    return pl.pallas_call(
        _vote_body,
        out_shape=[
            jax.ShapeDtypeStruct((_BATCH, 1), jnp.int32),
            jax.ShapeDtypeStruct((_BATCH, _N_CLASSES), jnp.float32),
        ],
    )(vals, labels)


def kernel(x, train_data, train_labels):
    td_t = jnp.pad(train_data, ((0, _NTP - _N_TRAIN), (0, 0)),
                   constant_values=1e18).T
    vals, idx = _run_topk(x, td_t)
    labels = _run_sc_gather(train_labels, idx)
    pred, proba = _run_vote(vals, labels)
    return (pred.reshape(_BATCH), proba)
